# Initial kernel scaffold; baseline (speedup 1.0000x reference)
import jax
import jax.numpy as jnp
from jax.experimental import pallas as pl


def kernel(layer_input, edge_index, edge_weight, W, b):
    raise NotImplementedError("write your pallas kernel here")


# speedup vs baseline: 4.3722x; 4.3722x over previous
import functools

import jax
import jax.numpy as jnp
from jax import lax
from jax.experimental import pallas as pl
from jax.experimental.pallas import tpu as pltpu
from jax.experimental.pallas import tpu_sc as plsc

_NC = 2
_NS = 16
_L = 16
_NW = _NC * _NS

_N = 10000
_E = 320000
_D = 128

_EW = _E // _NW
_CHUNK = 80
_NCHUNK = _EW // _CHUNK
_WB = 80
_NWB = _N // _WB
_WITER = -(-_NWB // _NS)
_VPR = _D // _L


def _mm_body(x_ref, wt_ref, b_ref, o_ref):
    o_ref[...] = (
        jnp.dot(x_ref[...], wt_ref[...], preferred_element_type=jnp.float32)
        + b_ref[...]
    )


def _linear(x, wt, b):
    m, k = x.shape
    n = wt.shape[1]
    bm = 1000
    return pl.pallas_call(
        _mm_body,
        grid=(m // bm,),
        in_specs=[
            pl.BlockSpec((bm, k), lambda i: (i, 0)),
            pl.BlockSpec((k, n), lambda i: (0, 0)),
            pl.BlockSpec((1, n), lambda i: (0, 0)),
        ],
        out_specs=pl.BlockSpec((bm, n), lambda i: (i, 0)),
        out_shape=jax.ShapeDtypeStruct((m, n), jnp.float32),
    )(x, wt, b.reshape(1, n))


def _add_body(a_ref, b_ref, o_ref):
    o_ref[...] = a_ref[...] + b_ref[...]


def _add(a, b):
    m, n = a.shape
    bm = 1000
    return pl.pallas_call(
        _add_body,
        grid=(m // bm,),
        in_specs=[
            pl.BlockSpec((bm, n), lambda i: (i, 0)),
            pl.BlockSpec((bm, n), lambda i: (i, 0)),
        ],
        out_specs=pl.BlockSpec((bm, n), lambda i: (i, 0)),
        out_shape=jax.ShapeDtypeStruct((m, n), jnp.float32),
    )(a, b)


def _sc_aggregate(x, src, dst, w):
    mesh = plsc.VectorSubcoreMesh(core_axis_name="c", subcore_axis_name="s")

    @functools.partial(
        pl.kernel,
        mesh=mesh,
        out_type=jax.ShapeDtypeStruct((_NC, _N, _D), jnp.float32),
        scratch_types=[
            pltpu.VMEM((_CHUNK,), jnp.int32),
            pltpu.VMEM((_CHUNK,), jnp.int32),
            pltpu.VMEM((_CHUNK,), jnp.float32),
            pltpu.VMEM((_CHUNK, _D), jnp.float32),
            pltpu.VMEM((_WB, _D), jnp.float32),
            pltpu.VMEM_SHARED((_N, _D), jnp.float32),
            pltpu.SemaphoreType.DMA,
        ],
    )
    def k(x_hbm, src_hbm, dst_hbm, w_hbm, out_hbm, src_v, dst_v, w_v, rows_v,
          zbuf, acc, sem):
        cid = lax.axis_index("c")
        sid = lax.axis_index("s")
        wid = sid * _NC + cid

        def zrow(i, _):
            for v in range(_VPR):
                zbuf[i, pl.ds(v * _L, _L)] = jnp.zeros((_L,), jnp.float32)
            return 0

        lax.fori_loop(0, _WB, zrow, 0)
        for i in range(_WITER):
            blk = sid + i * _NS

            @pl.when(blk < _NWB)
            def _():
                pltpu.sync_copy(zbuf, acc.at[pl.ds(blk * _WB, _WB)])

        plsc.subcore_barrier()

        def chunk_body(c, _):
            base = wid * _EW + c * _CHUNK
            pltpu.sync_copy(src_hbm.at[pl.ds(base, _CHUNK)], src_v)
            pltpu.sync_copy(dst_hbm.at[pl.ds(base, _CHUNK)], dst_v)
            pltpu.sync_copy(w_hbm.at[pl.ds(base, _CHUNK)], w_v)
            pltpu.async_copy(x_hbm.at[src_v], rows_v, sem).wait()

            def group_body(g, _):
                w16 = w_v[pl.ds(g * _L, _L)]
                for j in range(_L):
                    we = jnp.take_along_axis(
                        w16,
                        jnp.full((_L,), j, jnp.int32),
                        axis=0,
                        mode="promise_in_bounds",
                    )
                    e = g * _L + j
                    for v in range(_VPR):
                        rows_v[e, pl.ds(v * _L, _L)] = (
                            rows_v[e, pl.ds(v * _L, _L)] * we
                        )
                return 0

            lax.fori_loop(0, _CHUNK // _L, group_body, 0)
            pltpu.sync_copy(rows_v, acc.at[dst_v], add=True)
            return 0

        lax.fori_loop(0, _NCHUNK, chunk_body, 0)
        plsc.subcore_barrier()

        for i in range(_WITER):
            blk = sid + i * _NS

            @pl.when(blk < _NWB)
            def _():
                pltpu.sync_copy(acc.at[pl.ds(blk * _WB, _WB)], zbuf)
                pltpu.sync_copy(zbuf, out_hbm.at[cid, pl.ds(blk * _WB, _WB)])

    return k(x, src, dst, w)


def kernel(layer_input, edge_index, edge_weight, W, b):
    x = _linear(layer_input, W.T, b)
    src = edge_index[1]
    dst = edge_index[0]
    partials = _sc_aggregate(x, src, dst, edge_weight)
    return _add(partials[0], partials[1])


# speedup vs baseline: 9.9732x; 2.2810x over previous
import functools

import jax
import jax.numpy as jnp
from jax import lax
from jax.experimental import pallas as pl
from jax.experimental.pallas import tpu as pltpu
from jax.experimental.pallas import tpu_sc as plsc

_NC = 2
_NS = 16
_L = 16
_NW = _NC * _NS

_N = 10000
_E = 320000
_D = 128

_EW = _E // _NW
_CHUNK = 80
_NCHUNK = _EW // _CHUNK
_SEC = 2000
_NSEC = _EW // _SEC
_SCH = _SEC // _CHUNK
_DSTR = 32
_WB = 80
_NWB = _N // _WB
_WITER = -(-_NWB // _NS)
_VPR = _D // _L


def _mm_body(x_ref, wt_ref, b_ref, o_ref):
    o_ref[...] = (
        jnp.dot(x_ref[...], wt_ref[...], preferred_element_type=jnp.float32)
        + b_ref[...]
    )


def _linear(x, wt, b):
    m, k = x.shape
    n = wt.shape[1]
    bm = 1000
    return pl.pallas_call(
        _mm_body,
        grid=(m // bm,),
        in_specs=[
            pl.BlockSpec((bm, k), lambda i: (i, 0)),
            pl.BlockSpec((k, n), lambda i: (0, 0)),
            pl.BlockSpec((1, n), lambda i: (0, 0)),
        ],
        out_specs=pl.BlockSpec((bm, n), lambda i: (i, 0)),
        out_shape=jax.ShapeDtypeStruct((m, n), jnp.float32),
    )(x, wt, b.reshape(1, n))


def _add_body(a_ref, b_ref, o_ref):
    o_ref[...] = a_ref[...] + b_ref[...]


def _add(a, b):
    m, n = a.shape
    bm = 1000
    return pl.pallas_call(
        _add_body,
        grid=(m // bm,),
        in_specs=[
            pl.BlockSpec((bm, n), lambda i: (i, 0)),
            pl.BlockSpec((bm, n), lambda i: (i, 0)),
        ],
        out_specs=pl.BlockSpec((bm, n), lambda i: (i, 0)),
        out_shape=jax.ShapeDtypeStruct((m, n), jnp.float32),
    )(a, b)


def _sc_aggregate(x, src, dst, w):
    mesh = plsc.VectorSubcoreMesh(core_axis_name="c", subcore_axis_name="s")

    @functools.partial(
        pl.kernel,
        mesh=mesh,
        out_type=jax.ShapeDtypeStruct((_NC, _N, _D), jnp.float32),
        scratch_types=[
            pltpu.VMEM((2 * _SEC,), jnp.int32),
            pltpu.VMEM((2 * _DSTR, _CHUNK), jnp.int32),
            pltpu.VMEM((2 * _SEC,), jnp.float32),
            pltpu.VMEM((_CHUNK, _D), jnp.float32),
            pltpu.VMEM((_CHUNK, _D), jnp.float32),
            pltpu.VMEM_SHARED((_N, _D), jnp.float32),
            pltpu.SemaphoreType.DMA,
            pltpu.SemaphoreType.DMA,
            pltpu.SemaphoreType.DMA,
        ],
    )
    def k(x_hbm, src_hbm, dst_hbm, w_hbm, out_hbm, src_v, dst_v, w_v, rows0,
          rows1, acc, sem0, sem1, semi):
        zbuf = rows0
        cid = lax.axis_index("c")
        sid = lax.axis_index("s")
        wid = sid * _NC + cid

        def zrow(i, _):
            for v in range(_VPR):
                zbuf[i, pl.ds(v * _L, _L)] = jnp.zeros((_L,), jnp.float32)
            return 0

        lax.fori_loop(0, _WB, zrow, 0)
        for i in range(_WITER):
            blk = sid + i * _NS

            @pl.when(blk < _NWB)
            def _():
                pltpu.sync_copy(zbuf, acc.at[pl.ds(blk * _WB, _WB)])

        def start_idx_loads(s, sb):
            pltpu.async_copy(
                src_hbm.at[pl.ds(wid * _EW + s * _SEC, _SEC)],
                src_v.at[pl.ds(sb * _SEC, _SEC)], semi)
            pltpu.async_copy(
                w_hbm.at[pl.ds(wid * _EW + s * _SEC, _SEC)],
                w_v.at[pl.ds(sb * _SEC, _SEC)], semi)
            pltpu.async_copy(
                dst_hbm.at[wid * _NSEC + s],
                dst_v.at[pl.ds(sb * _DSTR, _SCH)], semi)

        def wait_idx_loads():
            pltpu.make_async_copy(
                src_hbm.at[pl.ds(0, _SEC)], src_v.at[pl.ds(0, _SEC)],
                semi).wait()
            pltpu.make_async_copy(
                w_hbm.at[pl.ds(0, _SEC)], w_v.at[pl.ds(0, _SEC)], semi).wait()
            pltpu.make_async_copy(
                dst_hbm.at[0], dst_v.at[pl.ds(0, _SCH)], semi).wait()

        def start_gather(sb, c, rows, sem):
            pltpu.async_copy(
                x_hbm.at[src_v.at[pl.ds(sb * _SEC + c * _CHUNK, _CHUNK)]],
                rows, sem)

        def wait_gather(rows, sem):
            pltpu.make_async_copy(
                x_hbm.at[src_v.at[pl.ds(0, _CHUNK)]], rows, sem
            ).wait()

        def scale(sb, c, rows):
            def group_body(g, _):
                w16 = w_v[pl.ds(sb * _SEC + c * _CHUNK + g * _L, _L)]
                for j in range(_L):
                    we = jnp.take_along_axis(
                        w16,
                        jnp.full((_L,), j, jnp.int32),
                        axis=0,
                        mode="promise_in_bounds",
                    )
                    e = g * _L + j
                    for v in range(_VPR):
                        rows[e, pl.ds(v * _L, _L)] = (
                            rows[e, pl.ds(v * _L, _L)] * we
                        )
                return 0

            lax.fori_loop(0, _CHUNK // _L, group_body, 0)

        def step(sb, c, rows, sem):
            wait_gather(rows, sem)
            scale(sb, c, rows)
            pltpu.sync_copy(rows, acc.at[dst_v.at[sb * _DSTR + c]], add=True)

            @pl.when(c + 2 < _SCH)
            def _():
                start_gather(sb, c + 2, rows, sem)

        start_idx_loads(0, 0)

        def section_body(s, _):
            sb = s % 2
            wait_idx_loads()

            @pl.when(s + 1 < _NSEC)
            def _():
                start_idx_loads(s + 1, 1 - sb)

            start_gather(sb, 0, rows0, sem0)
            start_gather(sb, 1, rows1, sem1)

            def pair_body(i, _):
                step(sb, 2 * i, rows0, sem0)
                step(sb, 2 * i + 1, rows1, sem1)
                return 0

            lax.fori_loop(0, _SCH // 2, pair_body, 0)
            if _SCH % 2:
                step(sb, _SCH - 1, rows0, sem0)
            return 0

        lax.fori_loop(0, _NSEC, section_body, 0)
        plsc.subcore_barrier()

        for i in range(_WITER):
            blk = sid + i * _NS

            @pl.when(blk < _NWB)
            def _():
                pltpu.sync_copy(acc.at[pl.ds(blk * _WB, _WB)], zbuf)
                pltpu.sync_copy(zbuf, out_hbm.at[cid, pl.ds(blk * _WB, _WB)])

    return k(x, src, dst, w)


def kernel(layer_input, edge_index, edge_weight, W, b):
    x = _linear(layer_input, W.T, b)
    src = edge_index[1]
    dst = edge_index[0].reshape(_NW * _NSEC, _SCH, _CHUNK)
    partials = _sc_aggregate(x, src, dst, edge_weight)
    return _add(partials[0], partials[1])


# speedup vs baseline: 11.0557x; 1.1085x over previous
import functools

import jax
import jax.numpy as jnp
from jax import lax
from jax.experimental import pallas as pl
from jax.experimental.pallas import tpu as pltpu
from jax.experimental.pallas import tpu_sc as plsc

_NC = 2
_NS = 16
_L = 16
_NW = _NC * _NS

_N = 10000
_E = 320000
_D = 128

_EW = _E // _NW
_CHUNK = 80
_NCHUNK = _EW // _CHUNK
_SEC = 2000
_NSEC = _EW // _SEC
_SCH = _SEC // _CHUNK
_DSTR = 32
_WB = 80
_NWB = _N // _WB
_WITER = -(-_NWB // _NS)
_VPR = _D // _L


def _mm_body(x_ref, wt_ref, b_ref, o_ref):
    o_ref[...] = (
        jnp.dot(x_ref[...], wt_ref[...], preferred_element_type=jnp.float32)
        + b_ref[...]
    )


def _linear(x, wt, b):
    m, k = x.shape
    n = wt.shape[1]
    bm = 1000
    return pl.pallas_call(
        _mm_body,
        grid=(m // bm,),
        in_specs=[
            pl.BlockSpec((bm, k), lambda i: (i, 0)),
            pl.BlockSpec((k, n), lambda i: (0, 0)),
            pl.BlockSpec((1, n), lambda i: (0, 0)),
        ],
        out_specs=pl.BlockSpec((bm, n), lambda i: (i, 0)),
        out_shape=jax.ShapeDtypeStruct((m, n), jnp.float32),
    )(x, wt, b.reshape(1, n))


def _add_body(a_ref, b_ref, o_ref):
    o_ref[...] = a_ref[...] + b_ref[...]


def _add(a, b):
    m, n = a.shape
    bm = 1000
    return pl.pallas_call(
        _add_body,
        grid=(m // bm,),
        in_specs=[
            pl.BlockSpec((bm, n), lambda i: (i, 0)),
            pl.BlockSpec((bm, n), lambda i: (i, 0)),
        ],
        out_specs=pl.BlockSpec((bm, n), lambda i: (i, 0)),
        out_shape=jax.ShapeDtypeStruct((m, n), jnp.float32),
    )(a, b)


def _sc_aggregate(x, src, dst, w):
    mesh = plsc.VectorSubcoreMesh(core_axis_name="c", subcore_axis_name="s")

    @functools.partial(
        pl.kernel,
        mesh=mesh,
        out_type=jax.ShapeDtypeStruct((_NC, _N, _D), jnp.float32),
        scratch_types=[
            pltpu.VMEM((2 * _SEC,), jnp.int32),
            pltpu.VMEM((2 * _DSTR, _CHUNK), jnp.int32),
            pltpu.VMEM((2 * _SEC,), jnp.float32),
            pltpu.VMEM((_CHUNK, _D), jnp.float32),
            pltpu.VMEM((_CHUNK, _D), jnp.float32),
            pltpu.VMEM((_CHUNK, _D), jnp.float32),
            pltpu.VMEM_SHARED((_N, _D), jnp.float32),
            pltpu.SemaphoreType.DMA,
            pltpu.SemaphoreType.DMA,
            pltpu.SemaphoreType.DMA,
            pltpu.SemaphoreType.DMA,
            pltpu.SemaphoreType.DMA,
            pltpu.SemaphoreType.DMA,
            pltpu.SemaphoreType.DMA,
        ],
    )
    def k(x_hbm, src_hbm, dst_hbm, w_hbm, out_hbm, src_v, dst_v, w_v, rows0,
          rows1, rows2, acc, semg0, semg1, semg2, sems0, sems1, sems2, semi):
        zbuf = rows0
        cid = lax.axis_index("c")
        sid = lax.axis_index("s")
        wid = sid * _NC + cid

        def zrow(i, _):
            for v in range(_VPR):
                zbuf[i, pl.ds(v * _L, _L)] = jnp.zeros((_L,), jnp.float32)
            return 0

        lax.fori_loop(0, _WB, zrow, 0)
        for i in range(_WITER):
            blk = sid + i * _NS

            @pl.when(blk < _NWB)
            def _():
                pltpu.sync_copy(zbuf, acc.at[pl.ds(blk * _WB, _WB)])

        def start_idx_loads(s, sb):
            pltpu.async_copy(
                src_hbm.at[pl.ds(wid * _EW + s * _SEC, _SEC)],
                src_v.at[pl.ds(sb * _SEC, _SEC)], semi)
            pltpu.async_copy(
                w_hbm.at[pl.ds(wid * _EW + s * _SEC, _SEC)],
                w_v.at[pl.ds(sb * _SEC, _SEC)], semi)
            pltpu.async_copy(
                dst_hbm.at[wid * _NSEC + s],
                dst_v.at[pl.ds(sb * _DSTR, _SCH)], semi)

        def wait_idx_loads():
            pltpu.make_async_copy(
                src_hbm.at[pl.ds(0, _SEC)], src_v.at[pl.ds(0, _SEC)],
                semi).wait()
            pltpu.make_async_copy(
                w_hbm.at[pl.ds(0, _SEC)], w_v.at[pl.ds(0, _SEC)], semi).wait()
            pltpu.make_async_copy(
                dst_hbm.at[0], dst_v.at[pl.ds(0, _SCH)], semi).wait()

        def start_gather(sb, c, rows, sem):
            pltpu.async_copy(
                x_hbm.at[src_v.at[pl.ds(sb * _SEC + c * _CHUNK, _CHUNK)]],
                rows, sem)

        def wait_gather(rows, sem):
            pltpu.make_async_copy(
                x_hbm.at[src_v.at[pl.ds(0, _CHUNK)]], rows, sem
            ).wait()

        def start_scatter(sb, c, rows, sem):
            pltpu.async_copy(
                rows, acc.at[dst_v.at[sb * _DSTR + c]], sem, add=True)

        def wait_scatter(rows, sem):
            pltpu.make_async_copy(
                rows, acc.at[dst_v.at[0]], sem).wait()

        def scale(sb, c, rows):
            def group_body(g, _):
                w16 = w_v[pl.ds(sb * _SEC + c * _CHUNK + g * _L, _L)]
                for j in range(_L):
                    we = jnp.take_along_axis(
                        w16,
                        jnp.full((_L,), j, jnp.int32),
                        axis=0,
                        mode="promise_in_bounds",
                    )
                    e = g * _L + j
                    for v in range(_VPR):
                        rows[e, pl.ds(v * _L, _L)] = (
                            rows[e, pl.ds(v * _L, _L)] * we
                        )
                return 0

            lax.fori_loop(0, _CHUNK // _L, group_body, 0)

        bufs = ((rows0, semg0, sems0), (rows1, semg1, sems1),
                (rows2, semg2, sems2))

        def step(sb, c, cur, prev, first):
            rows, semg, sems = cur
            prows, psemg, psems = prev
            wait_gather(rows, semg)
            scale(sb, c, rows)
            start_scatter(sb, c, rows, sems)
            if not first:
                wait_scatter(prows, psems)

            @pl.when(c + 2 < _SCH)
            def _():
                start_gather(sb, c + 2, prows, psemg)

        start_idx_loads(0, 0)

        def section_body(s, _):
            sb = s % 2
            wait_idx_loads()

            @pl.when(s + 1 < _NSEC)
            def _():
                start_idx_loads(s + 1, 1 - sb)

            @pl.when(s > 0)
            def _():
                wait_scatter(rows0, sems0)

            start_gather(sb, 0, rows0, semg0)
            start_gather(sb, 1, rows1, semg1)
            step(sb, 0, bufs[0], bufs[2], True)

            def tri_body(i, _):
                step(sb, 3 * i + 1, bufs[1], bufs[0], False)
                step(sb, 3 * i + 2, bufs[2], bufs[1], False)
                step(sb, 3 * i + 3, bufs[0], bufs[2], False)
                return 0

            lax.fori_loop(0, (_SCH - 1) // 3, tri_body, 0)
            return 0

        lax.fori_loop(0, _NSEC, section_body, 0)
        wait_scatter(rows0, sems0)
        plsc.subcore_barrier()

        for i in range(_WITER):
            blk = sid + i * _NS

            @pl.when(blk < _NWB)
            def _():
                pltpu.sync_copy(acc.at[pl.ds(blk * _WB, _WB)], zbuf)
                pltpu.sync_copy(zbuf, out_hbm.at[cid, pl.ds(blk * _WB, _WB)])

    return k(x, src, dst, w)


def kernel(layer_input, edge_index, edge_weight, W, b):
    x = _linear(layer_input, W.T, b)
    src = edge_index[1]
    dst = edge_index[0].reshape(_NW * _NSEC, _SCH, _CHUNK)
    partials = _sc_aggregate(x, src, dst, edge_weight)
    return _add(partials[0], partials[1])
